# baseline (speedup 1.0000x reference)
import jax
import jax.numpy as jnp
from jax import lax
from jax.experimental import pallas as pl
from jax.experimental.pallas import tpu as pltpu

_C = 64
_HB = 16
_CB = 2

_HP = jax.lax.Precision.DEFAULT
_F32 = jnp.float32


def _dot(a, b, prec=_HP):
    return jnp.dot(a, b, preferred_element_type=_F32, precision=prec)


def _dot_t(a, b, dims, prec=_HP):
    return lax.dot_general(a, b, (dims, ((), ())),
                           preferred_element_type=_F32, precision=prec)


def _gdn_kernel(kq_ref, v_ref, g_ref, b_ref, s0_ref, o_ref, fs_ref):
    c = pl.program_id(1)
    C = _C
    HB = _HB

    @pl.when(c == 0)
    def _init():
        fs_ref[...] = s0_ref[...]

    tt = lax.broadcasted_iota(jnp.int32, (C, C), 0)
    ii = lax.broadcasted_iota(jnp.int32, (C, C), 1)
    cum_mask = (tt <= ii).astype(_F32)
    eye = (tt == ii).astype(_F32)
    m_strict = (tt > ii).astype(_F32)
    m_inclT = (ii >= tt).astype(_F32)

    kqs = [[], []]
    Gcs, Gls, bcs = [[], []], [[], []], [[], []]
    egc2s, d_strict, d_inclT = [[], []], [[], []], [[], []]
    states = []
    for h in range(HB):
        states.append(fs_ref[h])
        for s in range(_CB):
            kqs[s].append(kq_ref[h, s])
            g = g_ref[h, s]
            beta = b_ref[h, s]
            G = _dot(g, cum_mask, prec=jax.lax.Precision.HIGHEST)
            Gc = G.reshape(C, 1)
            Gcs[s].append(Gc)
            Gls[s].append(G[:, C - 1:C])
            bcs[s].append(beta.reshape(C, 1))
            egc2s[s].append(jnp.exp(jnp.concatenate([Gc, Gc], axis=0)))
            Gdiff = Gc - G
            d_strict[s].append(m_strict * jnp.exp(Gdiff))
            d_inclT[s].append(m_inclT * jnp.exp(-Gdiff))

    KK = [[_dot_t(kqs[s][h][:C], kqs[s][h], ((1,), (1,))) for h in range(HB)]
          for s in range(_CB)]

    SPa = [egc2s[0][h] * _dot(kqs[0][h], states[h]) for h in range(HB)]

    Ts, Ps = [[], []], [[], []]
    for h in range(HB):
        for s in range(_CB):
            A = bcs[s][h] * d_strict[s][h] * KK[s][h][:, :C]
            Ts[s].append(eye - A)
            Ps[s].append(A)
    for _ in range(5):
        for h in range(HB):
            for s in range(_CB):
                Y = _dot(jnp.concatenate([Ts[s][h], Ps[s][h]], axis=0),
                         Ps[s][h])
                Ts[s][h] = Ts[s][h] + Y[:C]
                Ps[s][h] = Y[C:]

    Ua = []
    for h in range(HB):
        R = bcs[0][h] * (v_ref[h, :C] - SPa[h][:C])
        Ua.append(_dot(Ts[0][h], R))
    states1 = []
    for h in range(HB):
        attnT = d_inclT[0][h] * KK[0][h][:, C:]
        kd = kqs[0][h][:C] * jnp.exp(Gls[0][h] - Gcs[0][h])
        Z = _dot_t(jnp.concatenate([attnT, kd], axis=1), Ua[h],
                   ((0,), (0,)))
        o_ref[h, :C] = SPa[h][C:] + Z[:C]
        states1.append(jnp.exp(Gls[0][h]) * states[h] + Z[C:])

    SPb = [egc2s[1][h] * _dot(kqs[1][h], states1[h]) for h in range(HB)]
    Ub = []
    for h in range(HB):
        R = bcs[1][h] * (v_ref[h, C:] - SPb[h][:C])
        Ub.append(_dot(Ts[1][h], R))
    for h in range(HB):
        attnT = d_inclT[1][h] * KK[1][h][:, C:]
        kd = kqs[1][h][:C] * jnp.exp(Gls[1][h] - Gcs[1][h])
        Z = _dot_t(jnp.concatenate([attnT, kd], axis=1), Ub[h],
                   ((0,), (0,)))
        o_ref[h, C:] = SPb[h][C:] + Z[:C]
        fs_ref[h] = jnp.exp(Gls[1][h]) * states1[h] + Z[C:]


def kernel(query, key, value, g, beta, last_recurrent_state):
    B, H, S, D = query.shape
    BH = B * H
    C = _C
    HB = _HB
    CB = _CB
    NC = S // C
    NP = NC // CB
    NH = BH // HB

    kc = key.reshape(BH, NC, C, D)
    qc = query.reshape(BH, NC, C, D)
    kq = jnp.concatenate([kc, qc], axis=2)
    v = value.reshape(BH, S, D)
    g4 = g.reshape(BH, NC, 1, C)
    b4 = beta.reshape(BH, NC, 1, C)
    s0 = last_recurrent_state.reshape(BH, D, D)

    kq_spec = pl.BlockSpec((HB, CB, 2 * C, D), lambda h, c: (h, c, 0, 0))
    v_spec = pl.BlockSpec((HB, CB * C, D), lambda h, c: (h, c, 0))
    gb_spec = pl.BlockSpec((HB, CB, 1, C), lambda h, c: (h, c, 0, 0))
    st_spec = pl.BlockSpec((HB, D, D), lambda h, c: (h, 0, 0))

    o, fs = pl.pallas_call(
        _gdn_kernel,
        grid=(NH, NP),
        in_specs=[kq_spec, v_spec, gb_spec, gb_spec, st_spec],
        out_specs=[pl.BlockSpec((HB, CB * C, D), lambda h, c: (h, c, 0)),
                   st_spec],
        out_shape=[
            jax.ShapeDtypeStruct((BH, S, D), jnp.float32),
            jax.ShapeDtypeStruct((BH, D, D), jnp.float32),
        ],
        compiler_params=pltpu.CompilerParams(
            dimension_semantics=("parallel", "arbitrary"),
        ),
        name="gdn_chunked",
    )(kq, v, g4, b4, s0)

    return jnp.concatenate([o.reshape(-1), fs.reshape(-1)], axis=0)


# speedup vs baseline: 1.0511x; 1.0511x over previous
import jax
import jax.numpy as jnp
from jax import lax
from jax.experimental import pallas as pl
from jax.experimental.pallas import tpu as pltpu

_C = 64
_HB = 32

_HP = jax.lax.Precision.DEFAULT
_F32 = jnp.float32


def _dot(a, b, prec=_HP):
    return jnp.dot(a, b, preferred_element_type=_F32, precision=prec)


def _dot_t(a, b, dims, prec=_HP):
    return lax.dot_general(a, b, (dims, ((), ())),
                           preferred_element_type=_F32, precision=prec)


def _gdn_kernel(kq_ref, v_ref, g_ref, b_ref, s0_ref, o_ref, fs_ref):
    c = pl.program_id(1)
    C = _C
    HB = _HB

    @pl.when(c == 0)
    def _init():
        fs_ref[...] = s0_ref[...]

    tt = lax.broadcasted_iota(jnp.int32, (C, C), 0)
    ii = lax.broadcasted_iota(jnp.int32, (C, C), 1)
    cum_mask = (tt <= ii).astype(_F32)
    eye = (tt == ii).astype(_F32)
    m_strict = (tt > ii).astype(_F32)
    m_inclT = (ii >= tt).astype(_F32)

    kqs, vs, states = [], [], []
    Gcs, Gls, bcs, egc2s = [], [], [], []
    d_strict, d_inclT = [], []
    for h in range(HB):
        kqs.append(kq_ref[h, 0])
        vs.append(v_ref[h])
        states.append(fs_ref[h])
        g = g_ref[h, 0]
        beta = b_ref[h, 0]
        G = _dot(g, cum_mask, prec=jax.lax.Precision.HIGHEST)
        Gc = G.reshape(C, 1)
        Gcs.append(Gc)
        Gls.append(G[:, C - 1:C])
        bcs.append(beta.reshape(C, 1))
        egc2s.append(jnp.exp(jnp.concatenate([Gc, Gc], axis=0)))
        Gdiff = Gc - G
        d_strict.append(m_strict * jnp.exp(Gdiff))
        d_inclT.append(m_inclT * jnp.exp(-Gdiff))

    KK = [_dot_t(kqs[h][:C], kqs[h], ((1,), (1,))) for h in range(HB)]

    SP = [egc2s[h] * _dot(kqs[h], states[h]) for h in range(HB)]

    Ts, Ps = [], []
    for h in range(HB):
        A = bcs[h] * d_strict[h] * KK[h][:, :C]
        Ts.append(eye - A)
        Ps.append(A)
    for _ in range(5):
        for h in range(HB):
            Y = _dot(jnp.concatenate([Ts[h], Ps[h]], axis=0), Ps[h])
            Ts[h] = Ts[h] + Y[:C]
            Ps[h] = Y[C:]

    Us = []
    for h in range(HB):
        R = bcs[h] * (vs[h] - SP[h][:C])
        Us.append(_dot(Ts[h], R))

    for h in range(HB):
        attnT = d_inclT[h] * KK[h][:, C:]
        kd = kqs[h][:C] * jnp.exp(Gls[h] - Gcs[h])
        Z = _dot_t(jnp.concatenate([attnT, kd], axis=1), Us[h],
                   ((0,), (0,)))
        o_ref[h] = SP[h][C:] + Z[:C]
        fs_ref[h] = jnp.exp(Gls[h]) * states[h] + Z[C:]


def kernel(query, key, value, g, beta, last_recurrent_state):
    B, H, S, D = query.shape
    BH = B * H
    C = _C
    HB = _HB
    NC = S // C
    NH = BH // HB

    kc = key.reshape(BH, NC, C, D)
    qc = query.reshape(BH, NC, C, D)
    kq = jnp.concatenate([kc, qc], axis=2)
    v = value.reshape(BH, S, D)
    g4 = g.reshape(BH, NC, 1, C)
    b4 = beta.reshape(BH, NC, 1, C)
    s0 = last_recurrent_state.reshape(BH, D, D)

    kq_spec = pl.BlockSpec((HB, 1, 2 * C, D), lambda h, c: (h, c, 0, 0))
    v_spec = pl.BlockSpec((HB, C, D), lambda h, c: (h, c, 0))
    gb_spec = pl.BlockSpec((HB, 1, 1, C), lambda h, c: (h, c, 0, 0))
    st_spec = pl.BlockSpec((HB, D, D), lambda h, c: (h, 0, 0))

    o, fs = pl.pallas_call(
        _gdn_kernel,
        grid=(NH, NC),
        in_specs=[kq_spec, v_spec, gb_spec, gb_spec, st_spec],
        out_specs=[pl.BlockSpec((HB, C, D), lambda h, c: (h, c, 0)), st_spec],
        out_shape=[
            jax.ShapeDtypeStruct((BH, S, D), jnp.float32),
            jax.ShapeDtypeStruct((BH, D, D), jnp.float32),
        ],
        compiler_params=pltpu.CompilerParams(
            dimension_semantics=("parallel", "arbitrary"),
        ),
        name="gdn_chunked",
    )(kq, v, g4, b4, s0)

    return jnp.concatenate([o.reshape(-1), fs.reshape(-1)], axis=0)


# speedup vs baseline: 1.2681x; 1.2065x over previous
import jax
import jax.numpy as jnp
from jax import lax
from jax.experimental import pallas as pl
from jax.experimental.pallas import tpu as pltpu

_C = 64
_HB = 32

_HP = jax.lax.Precision.DEFAULT
_F32 = jnp.float32


def _dot(a, b, prec=_HP):
    return jnp.dot(a, b, preferred_element_type=_F32, precision=prec)


def _dot_t(a, b, dims, prec=_HP):
    return lax.dot_general(a, b, (dims, ((), ())),
                           preferred_element_type=_F32, precision=prec)


def _gdn_kernel(q_ref, k_ref, v_ref, g_ref, b_ref, s0_ref, o_ref, fs_ref):
    c = pl.program_id(1)
    C = _C
    HB = _HB

    @pl.when(c == 0)
    def _init():
        fs_ref[...] = s0_ref[...]

    tt = lax.broadcasted_iota(jnp.int32, (C, C), 0)
    ii = lax.broadcasted_iota(jnp.int32, (C, C), 1)
    cum_mask = (tt <= ii).astype(_F32)
    eye = (tt == ii).astype(_F32)
    m_strict = (tt > ii).astype(_F32)
    m_inclT = (ii >= tt).astype(_F32)

    ks, qs, vs, states = [], [], [], []
    Gcs, Gls, bcs = [], [], []
    d_strict, d_inclT = [], []
    for h in range(HB):
        ks.append(k_ref[h])
        qs.append(q_ref[h])
        vs.append(v_ref[h])
        states.append(fs_ref[h])
        g = g_ref[h, 0]
        beta = b_ref[h, 0]
        G = _dot(g, cum_mask, prec=jax.lax.Precision.HIGHEST)
        Gc = G.reshape(C, 1)
        Gcs.append(Gc)
        Gls.append(G[:, C - 1:C])
        bcs.append(beta.reshape(C, 1))
        Gdiff = Gc - G
        d_strict.append(m_strict * jnp.exp(Gdiff))
        d_inclT.append(m_inclT * jnp.exp(-Gdiff))

    KK = [_dot_t(ks[h], jnp.concatenate([ks[h], qs[h]], axis=0), ((1,), (1,)))
          for h in range(HB)]

    SP = [_dot(jnp.concatenate([ks[h] * jnp.exp(Gcs[h]),
                                qs[h] * jnp.exp(Gcs[h])], axis=0), states[h])
          for h in range(HB)]

    Ts, Ps = [], []
    for h in range(HB):
        A = bcs[h] * d_strict[h] * KK[h][:, :C]
        Ts.append(eye - A)
        Ps.append(A)
    for _ in range(5):
        for h in range(HB):
            Y = _dot(jnp.concatenate([Ts[h], Ps[h]], axis=0), Ps[h])
            Ts[h] = Ts[h] + Y[:C]
            Ps[h] = Y[C:]

    Us = []
    for h in range(HB):
        R = bcs[h] * (vs[h] - SP[h][:C])
        Us.append(_dot(Ts[h], R))

    for h in range(HB):
        attnT = d_inclT[h] * KK[h][:, C:]
        kd = ks[h] * jnp.exp(Gls[h] - Gcs[h])
        Z = _dot_t(jnp.concatenate([attnT, kd], axis=1), Us[h],
                   ((0,), (0,)))
        o_ref[h] = SP[h][C:] + Z[:C]
        fs_ref[h] = jnp.exp(Gls[h]) * states[h] + Z[C:]


def kernel(query, key, value, g, beta, last_recurrent_state):
    B, H, S, D = query.shape
    BH = B * H
    C = _C
    HB = _HB
    NC = S // C
    NH = BH // HB

    q = query.reshape(BH, S, D)
    k = key.reshape(BH, S, D)
    v = value.reshape(BH, S, D)
    g4 = g.reshape(BH, NC, 1, C)
    b4 = beta.reshape(BH, NC, 1, C)
    s0 = last_recurrent_state.reshape(BH, D, D)

    qkv_spec = pl.BlockSpec((HB, C, D), lambda h, c: (h, c, 0))
    gb_spec = pl.BlockSpec((HB, 1, 1, C), lambda h, c: (h, c, 0, 0))
    st_spec = pl.BlockSpec((HB, D, D), lambda h, c: (h, 0, 0))

    o, fs = pl.pallas_call(
        _gdn_kernel,
        grid=(NH, NC),
        in_specs=[qkv_spec, qkv_spec, qkv_spec, gb_spec, gb_spec, st_spec],
        out_specs=[pl.BlockSpec((HB, C, D), lambda h, c: (h, c, 0)), st_spec],
        out_shape=[
            jax.ShapeDtypeStruct((BH, S, D), jnp.float32),
            jax.ShapeDtypeStruct((BH, D, D), jnp.float32),
        ],
        compiler_params=pltpu.CompilerParams(
            dimension_semantics=("parallel", "arbitrary"),
        ),
        name="gdn_chunked",
    )(q, k, v, g4, b4, s0)

    return jnp.concatenate([o.reshape(-1), fs.reshape(-1)], axis=0)
